# baseline (speedup 1.0000x reference)
import functools

import jax
import jax.numpy as jnp
from jax.experimental import pallas as pl
from jax.experimental.pallas import tpu as pltpu

N = 16384
H = 2048
D = 64


def _fused_kernel(nk, nm, tk, tm, tar_hbm, embs_hbm, src_hbm, o_hbm,
                  embs_v, acc, tbuf, sbuf, obuf, esem, tsem, ssem, osem):
    def tar_copy(j, slot):
        return pltpu.make_async_copy(
            tar_hbm.at[:, pl.ds(j * tk, tk)], tbuf.at[slot], tsem.at[slot])

    def src_copy(m, slot):
        return pltpu.make_async_copy(
            src_hbm.at[pl.ds(m * tm, tm), :], sbuf.at[slot], ssem.at[slot])

    def out_copy(m, slot):
        return pltpu.make_async_copy(
            obuf.at[slot], o_hbm.at[pl.ds(m * tm, tm), :], osem.at[slot])

    ecopy = pltpu.make_async_copy(embs_hbm, embs_v, esem)
    tar_copy(0, 0).start()
    ecopy.start()
    tar_copy(1, 1).start()
    acc[...] = jnp.zeros_like(acc)
    ecopy.wait()

    def phase1(j, carry):
        slot = jax.lax.rem(j, 2)
        tar_copy(j, slot).wait()
        acc[...] += jnp.dot(tbuf[slot], embs_v[pl.ds(j * tk, tk), :],
                            preferred_element_type=jnp.float32)

        @pl.when(j + 2 < nk)
        def _next():
            tar_copy(j + 2, slot).start()

        @pl.when(j == nk - 2)
        def _warm0():
            src_copy(0, 0).start()

        @pl.when(j == nk - 1)
        def _warm1():
            src_copy(1, 1).start()

        return carry

    jax.lax.fori_loop(0, nk, phase1, 0)

    def phase2(m, carry):
        slot = jax.lax.rem(m, 2)
        src_copy(m, slot).wait()

        @pl.when(m >= 2)
        def _drain():
            out_copy(m - 2, slot).wait()

        obuf[slot] = jnp.maximum(
            jnp.dot(sbuf[slot], acc[...], preferred_element_type=jnp.float32),
            0.0)
        out_copy(m, slot).start()

        @pl.when(m + 2 < nm)
        def _next():
            src_copy(m + 2, slot).start()

        return carry

    jax.lax.fori_loop(0, nm, phase2, 0)
    out_copy(nm - 2, 0).wait()
    out_copy(nm - 1, 1).wait()


@functools.partial(jax.jit, static_argnames=("tk", "tm"))
def _run(pois_embs, HG_poi_src, HG_poi_tar, tk=1024, tm=1024):
    nk = N // tk
    nm = N // tm
    any_spec = pl.BlockSpec(memory_space=pltpu.MemorySpace.HBM)
    return pl.pallas_call(
        functools.partial(_fused_kernel, nk, nm, tk, tm),
        in_specs=[any_spec, any_spec, any_spec],
        out_specs=any_spec,
        out_shape=jax.ShapeDtypeStruct((N, D), jnp.float32),
        scratch_shapes=[
            pltpu.VMEM((N, D), jnp.float32),
            pltpu.VMEM((H, D), jnp.float32),
            pltpu.VMEM((2, H, tk), jnp.float32),
            pltpu.VMEM((2, tm, H), jnp.float32),
            pltpu.VMEM((2, tm, D), jnp.float32),
            pltpu.SemaphoreType.DMA,
            pltpu.SemaphoreType.DMA((2,)),
            pltpu.SemaphoreType.DMA((2,)),
            pltpu.SemaphoreType.DMA((2,)),
        ],
        compiler_params=pltpu.CompilerParams(
            vmem_limit_bytes=63 * 1024 * 1024),
    )(HG_poi_tar, pois_embs, HG_poi_src)


def kernel(pois_embs, HG_poi_src, HG_poi_tar):
    return _run(pois_embs, HG_poi_src, HG_poi_tar)


# speedup vs baseline: 1.0015x; 1.0015x over previous
import functools

import jax
import jax.numpy as jnp
from jax.experimental import pallas as pl
from jax.experimental.pallas import tpu as pltpu

N = 16384
H = 2048
D = 64


def _fused_kernel(nk, nm, tk, tm, tar_hbm, embs_hbm, src_hbm, o_hbm,
                  embs_v, acc, tbuf, sbuf, obuf,
                  esem, t0sem, tsem, ssem, osem):
    nsub = 4
    rsub = H // nsub

    def embs_copy(j):
        return pltpu.make_async_copy(
            embs_hbm.at[pl.ds(j * tk, tk), :],
            embs_v.at[pl.ds(j * tk, tk), :], esem.at[j])

    def tar0_copy(r):
        return pltpu.make_async_copy(
            tar_hbm.at[pl.ds(r * rsub, rsub), pl.ds(0, tk)],
            tbuf.at[0, pl.ds(r * rsub, rsub)], t0sem.at[r])

    def tar_copy(j, slot):
        return pltpu.make_async_copy(
            tar_hbm.at[:, pl.ds(j * tk, tk)], tbuf.at[slot], tsem.at[slot])

    def src_copy(m, slot):
        return pltpu.make_async_copy(
            src_hbm.at[pl.ds(m * tm, tm), :], sbuf.at[slot], ssem.at[slot])

    def out_copy(m, slot):
        return pltpu.make_async_copy(
            obuf.at[slot], o_hbm.at[pl.ds(m * tm, tm), :], osem.at[slot])

    embs_copy(0).start()
    for r in range(nsub):
        tar0_copy(r).start()
    embs_copy(1).start()
    tar_copy(1, 1).start()
    embs_copy(2).start()

    embs_copy(0).wait()
    for r in range(nsub):
        tar0_copy(r).wait()
        acc[pl.ds(r * rsub, rsub), :] = jnp.dot(
            tbuf[0, pl.ds(r * rsub, rsub)], embs_v[pl.ds(0, tk), :],
            preferred_element_type=jnp.float32)

    tar_copy(2, 0).start()

    def phase1(j, carry):
        slot = jax.lax.rem(j, 2)
        tar_copy(j, slot).wait()
        embs_copy(j).wait()
        acc[...] += jnp.dot(tbuf[slot], embs_v[pl.ds(j * tk, tk), :],
                            preferred_element_type=jnp.float32)

        @pl.when(j + 2 < nk)
        def _next():
            tar_copy(j + 2, slot).start()
            embs_copy(j + 2).start()

        @pl.when(j == nk - 2)
        def _warm0():
            src_copy(0, 0).start()

        @pl.when(j == nk - 1)
        def _warm1():
            src_copy(1, 1).start()

        return carry

    jax.lax.fori_loop(1, nk, phase1, 0)

    def phase2(m, carry):
        slot = jax.lax.rem(m, 2)
        src_copy(m, slot).wait()

        @pl.when(m >= 2)
        def _drain():
            out_copy(m - 2, slot).wait()

        obuf[slot] = jnp.maximum(
            jnp.dot(sbuf[slot], acc[...], preferred_element_type=jnp.float32),
            0.0)
        out_copy(m, slot).start()

        @pl.when(m + 2 < nm)
        def _next():
            src_copy(m + 2, slot).start()

        return carry

    jax.lax.fori_loop(0, nm, phase2, 0)
    out_copy(nm - 2, 0).wait()
    out_copy(nm - 1, 1).wait()


@functools.partial(jax.jit, static_argnames=("tk", "tm"))
def _run(pois_embs, HG_poi_src, HG_poi_tar, tk=1024, tm=1024):
    nk = N // tk
    nm = N // tm
    any_spec = pl.BlockSpec(memory_space=pltpu.MemorySpace.HBM)
    return pl.pallas_call(
        functools.partial(_fused_kernel, nk, nm, tk, tm),
        in_specs=[any_spec, any_spec, any_spec],
        out_specs=any_spec,
        out_shape=jax.ShapeDtypeStruct((N, D), jnp.float32),
        scratch_shapes=[
            pltpu.VMEM((N, D), jnp.float32),
            pltpu.VMEM((H, D), jnp.float32),
            pltpu.VMEM((2, H, tk), jnp.float32),
            pltpu.VMEM((2, tm, H), jnp.float32),
            pltpu.VMEM((2, tm, D), jnp.float32),
            pltpu.SemaphoreType.DMA((nk,)),
            pltpu.SemaphoreType.DMA((4,)),
            pltpu.SemaphoreType.DMA((2,)),
            pltpu.SemaphoreType.DMA((2,)),
            pltpu.SemaphoreType.DMA((2,)),
        ],
        compiler_params=pltpu.CompilerParams(
            vmem_limit_bytes=63 * 1024 * 1024),
    )(HG_poi_tar, pois_embs, HG_poi_src)


def kernel(pois_embs, HG_poi_src, HG_poi_tar):
    return _run(pois_embs, HG_poi_src, HG_poi_tar)
